# baseline (speedup 1.0000x reference)
import jax
import jax.numpy as jnp
from jax import lax
from jax.experimental import pallas as pl
from jax.experimental.pallas import tpu as pltpu
from jax.experimental.pallas import tpu_sc as plsc

B = 1024
BS = 15
FD = 64
PC = 2380
ED = 2 * (PC + 1)
CELLS = BS * BS
CPAD = 256


def _sc_kernel(pre_hbm, big_hbm, small_hbm, out_hbm,
               pre_v, idx_v, gbuf, out_t, sem):
    info = plsc.get_sparse_core_info()
    nc = info.num_cores
    wid = lax.axis_index("s") * nc + lax.axis_index("c")
    bpw = B // (nc * info.num_subcores)
    b_base = wid * bpw

    fidx = [(lax.iota(jnp.int32, 16) + fg * 16) * CELLS for fg in range(4)]

    def per_batch(i, carry):
        b = b_base + i
        pltpu.sync_copy(pre_hbm.at[b], pre_v)

        for g in range(16):
            c_vec = lax.iota(jnp.int32, 16) + (g * 16)
            c_eff = jnp.minimum(c_vec, CELLS - 1)
            s0 = pre_v[pl.ds(g * 16, 16)]
            s1 = pre_v[pl.ds(CPAD + g * 16, 16)]
            bd0 = pre_v[pl.ds(2 * CPAD + g * 16, 16)]
            bd1 = pre_v[pl.ds(3 * CPAD + g * 16, 16)]
            psi0 = jnp.where(bd0 > 0, jnp.int32(PC), s0)
            psi1 = jnp.where(bd1 > 0, jnp.int32(PC), s1) + jnp.int32(PC + 1)
            base = c_eff * ED
            row, off = g // 8, (g % 8) * 16
            idx_v[row, pl.ds(off, 16)] = psi0 + base
            idx_v[2 + row, pl.ds(off, 16)] = psi1 + base
            idx_v[4 + row, pl.ds(off, 16)] = psi0
            idx_v[6 + row, pl.ds(off, 16)] = psi1

        copies = [pltpu.async_copy(big_hbm.at[idx_v.at[k]], gbuf.at[k], sem)
                  for k in range(4)]
        copies += [pltpu.async_copy(small_hbm.at[idx_v.at[k]], gbuf.at[k], sem)
                   for k in range(4, 8)]
        for cp in copies:
            cp.wait()

        for half, n_r in ((0, 128), (1, CELLS - 128)):
            def tr_body(r, cr, half=half):
                c = half * 128 + r
                for fg in range(4):
                    fs = pl.ds(fg * 16, 16)
                    v = (gbuf[half, r, fs] + gbuf[2 + half, r, fs]
                         + gbuf[4 + half, r, fs] + gbuf[6 + half, r, fs])
                    plsc.store_scatter(out_t, [fidx[fg] + c], v)
                return cr
            lax.fori_loop(0, n_r, tr_body, 0)

        pltpu.sync_copy(out_t, out_hbm.at[b])
        return carry

    lax.fori_loop(0, bpw, per_batch, 0)


def kernel(sparse_feature_dim, sparse_feature_input, board_input, pcode_table, pcode_board_table):
    del sparse_feature_dim
    s = sparse_feature_input[:, 10:12].reshape(B, 2, CELLS)
    bd = board_input.reshape(B, 2, CELLS)
    packed = jnp.pad(jnp.concatenate([s, bd], axis=1),
                     ((0, 0), (0, 0), (0, CPAD - CELLS))).reshape(B, 4 * CPAD)

    mesh = plsc.VectorSubcoreMesh(core_axis_name="c", subcore_axis_name="s")
    out = pl.kernel(
        _sc_kernel,
        out_type=jax.ShapeDtypeStruct((B, FD * CELLS), jnp.float32),
        mesh=mesh,
        compiler_params=pltpu.CompilerParams(
            needs_layout_passes=False, use_tc_tiling_on_sc=False),
        scratch_types=[
            pltpu.VMEM((4 * CPAD,), jnp.int32),
            pltpu.VMEM((8, 128), jnp.int32),
            pltpu.VMEM((8, 128, FD), jnp.float32),
            pltpu.VMEM((FD * CELLS,), jnp.float32),
            pltpu.SemaphoreType.DMA,
        ],
    )(packed, pcode_board_table, pcode_table)
    return out.reshape(B, FD, BS, BS)


# speedup vs baseline: 2.1560x; 2.1560x over previous
import jax
import jax.numpy as jnp
from jax import lax
from jax.experimental import pallas as pl
from jax.experimental.pallas import tpu as pltpu
from jax.experimental.pallas import tpu_sc as plsc

B = 1024
BS = 15
FD = 64
PC = 2380
ED = 2 * (PC + 1)
CELLS = BS * BS
CPAD = 256


def _sc_kernel(pre_hbm, big_hbm, small_hbm, out_hbm,
               pre_v, idx_v, idx_s, gbuf, mpan, smrow, mskb, out_t, sem):
    info = plsc.get_sparse_core_info()
    nc = info.num_cores
    wid = lax.axis_index("s") * nc + lax.axis_index("c")
    bpw = B // (nc * info.num_subcores)
    b_base = wid * bpw

    iota = lax.iota(jnp.int32, 16)
    fidx = [(iota + fg * 16) * CELLS for fg in range(4)]

    for g in range(16):
        c_eff = jnp.minimum(iota + g * 16, CELLS - 1)
        row, off = g // 8, (g % 8) * 16
        idx_v[row, pl.ds(off, 16)] = c_eff * ED + jnp.int32(PC)
        idx_v[2 + row, pl.ds(off, 16)] = c_eff * ED + jnp.int32(2 * PC + 1)
    idx_s[pl.ds(0, 16)] = jnp.where(iota < 1, jnp.int32(PC),
                                    jnp.where(iota < 2, jnp.int32(2 * PC + 1), iota))
    stage = [pltpu.async_copy(big_hbm.at[idx_v.at[0]], mpan.at[0].at[pl.ds(0, 128)], sem),
             pltpu.async_copy(big_hbm.at[idx_v.at[1]], mpan.at[0].at[pl.ds(128, 128)], sem),
             pltpu.async_copy(big_hbm.at[idx_v.at[2]], mpan.at[1].at[pl.ds(0, 128)], sem),
             pltpu.async_copy(big_hbm.at[idx_v.at[3]], mpan.at[1].at[pl.ds(128, 128)], sem),
             pltpu.async_copy(small_hbm.at[idx_s], smrow, sem)]
    for cp in stage:
        cp.wait()
    sm = [[smrow[ch, pl.ds(fg * 16, 16)] for fg in range(4)] for ch in range(2)]

    def fold_body(j, carry):
        for ch in range(2):
            for fg in range(4):
                fs = pl.ds(fg * 16, 16)
                mpan[ch, j, fs] = mpan[ch, j, fs] + sm[ch][fg]
        return carry
    lax.fori_loop(0, CELLS, fold_body, 0)

    def per_batch(i, carry):
        b = b_base + i
        pltpu.sync_copy(pre_hbm.at[b], pre_v)

        for g in range(16):
            s0 = pre_v[pl.ds(g * 16, 16)]
            s1 = pre_v[pl.ds(CPAD + g * 16, 16)] + jnp.int32(PC + 1)
            base = jnp.minimum(iota + g * 16, CELLS - 1) * ED
            row, off = g // 8, (g % 8) * 16
            idx_v[row, pl.ds(off, 16)] = s0 + base
            idx_v[2 + row, pl.ds(off, 16)] = s1 + base
            idx_v[4 + row, pl.ds(off, 16)] = s0
            idx_v[6 + row, pl.ds(off, 16)] = s1
            if g < 15:
                bd0 = pre_v[pl.ds(2 * CPAD + g * 16, 16)]
                bd1 = pre_v[pl.ds(3 * CPAD + g * 16, 16)]
                mv0 = jnp.where(bd0 > 0, jnp.float32(1.0), jnp.float32(0.0))
                mv1 = jnp.where(bd1 > 0, jnp.float32(1.0), jnp.float32(0.0))
                cbase = (iota + g * 16) * 16
                for l in range(16):
                    plsc.store_scatter(mskb.at[0], [cbase + l], mv0)
                    plsc.store_scatter(mskb.at[1], [cbase + l], mv1)

        copies = [pltpu.async_copy(big_hbm.at[idx_v.at[k]], gbuf.at[k], sem)
                  for k in range(4)]
        copies += [pltpu.async_copy(small_hbm.at[idx_v.at[k]], gbuf.at[k], sem)
                   for k in range(4, 8)]
        for cp in copies:
            cp.wait()

        for half, n_r in ((0, 128), (1, CELLS - 128)):
            def tr_body(r, cr, half=half):
                c = half * 128 + r
                m0 = mskb[0, pl.ds(c * 16, 16)] > jnp.float32(0.5)
                m1 = mskb[1, pl.ds(c * 16, 16)] > jnp.float32(0.5)
                for fg in range(4):
                    fs = pl.ds(fg * 16, 16)
                    v0 = jnp.where(m0, mpan[0, c, fs],
                                   gbuf[half, r, fs] + gbuf[4 + half, r, fs])
                    v1 = jnp.where(m1, mpan[1, c, fs],
                                   gbuf[2 + half, r, fs] + gbuf[6 + half, r, fs])
                    plsc.store_scatter(out_t, [fidx[fg] + c], v0 + v1)
                return cr
            lax.fori_loop(0, n_r, tr_body, 0)

        pltpu.sync_copy(out_t, out_hbm.at[b])
        return carry

    lax.fori_loop(0, bpw, per_batch, 0)


def kernel(sparse_feature_dim, sparse_feature_input, board_input, pcode_table, pcode_board_table):
    del sparse_feature_dim
    s = sparse_feature_input[:, 10:12].reshape(B, 2, CELLS)
    bd = board_input.reshape(B, 2, CELLS)
    packed = jnp.pad(jnp.concatenate([s, bd], axis=1),
                     ((0, 0), (0, 0), (0, CPAD - CELLS))).reshape(B, 4 * CPAD)

    mesh = plsc.VectorSubcoreMesh(core_axis_name="c", subcore_axis_name="s")
    out = pl.kernel(
        _sc_kernel,
        out_type=jax.ShapeDtypeStruct((B, FD * CELLS), jnp.float32),
        mesh=mesh,
        compiler_params=pltpu.CompilerParams(
            needs_layout_passes=False, use_tc_tiling_on_sc=False),
        scratch_types=[
            pltpu.VMEM((4 * CPAD,), jnp.int32),
            pltpu.VMEM((8, 128), jnp.int32),
            pltpu.VMEM((16,), jnp.int32),
            pltpu.VMEM((8, 128, FD), jnp.float32),
            pltpu.VMEM((2, CPAD, FD), jnp.float32),
            pltpu.VMEM((16, FD), jnp.float32),
            pltpu.VMEM((2, 16 * (CPAD - 16)), jnp.float32),
            pltpu.VMEM((FD * CELLS,), jnp.float32),
            pltpu.SemaphoreType.DMA,
        ],
    )(packed, pcode_board_table, pcode_table)
    return out.reshape(B, FD, BS, BS)


# speedup vs baseline: 3.3191x; 1.5394x over previous
import jax
import jax.numpy as jnp
from jax import lax
from jax.experimental import pallas as pl
from jax.experimental.pallas import tpu as pltpu
from jax.experimental.pallas import tpu_sc as plsc

B = 1024
BS = 15
FD = 64
PC = 2380
ED = 2 * (PC + 1)
CELLS = BS * BS
CPAD = 256
NB = 112
SLAB = FD * CELLS


def _sc_kernel(pre_hbm, big_hbm, small_hbm, out_hbm,
               pre2, idxA, idxB, idx_s, gbufA, gbufB, mpan, mskb, out2,
               gsemA, gsemB, psem, osem):
    info = plsc.get_sparse_core_info()
    nc = info.num_cores
    wid = lax.axis_index("s") * nc + lax.axis_index("c")
    bpw = B // (nc * info.num_subcores)
    b_base = wid * bpw

    iota = lax.iota(jnp.int32, 16)
    fidx = [(iota + fg * 16) * CELLS for fg in range(4)]

    for g in range(8):
        c = iota + g * 16
        idxA[0, pl.ds(g * 16, 16)] = c * ED + jnp.int32(PC)
        idxA[1, pl.ds(g * 16, 16)] = c * ED + jnp.int32(2 * PC + 1)
    for g in range(7):
        c = iota + (g + 8) * 16
        c_eff = jnp.where(c < CELLS, c, c - 128)
        idxB[0, pl.ds(g * 16, 16)] = c_eff * ED + jnp.int32(PC)
        idxB[1, pl.ds(g * 16, 16)] = c_eff * ED + jnp.int32(2 * PC + 1)
    stage = [pltpu.async_copy(big_hbm.at[idxA.at[0]], mpan.at[0].at[pl.ds(0, 128)], gsemA),
             pltpu.async_copy(big_hbm.at[idxA.at[1]], mpan.at[1].at[pl.ds(0, 128)], gsemA),
             pltpu.async_copy(big_hbm.at[idxB.at[0]], mpan.at[0].at[pl.ds(128, NB)], gsemB),
             pltpu.async_copy(big_hbm.at[idxB.at[1]], mpan.at[1].at[pl.ds(128, NB)], gsemB)]
    idx_s[pl.ds(0, 16)] = jnp.where(iota < 1, jnp.int32(PC),
                                    jnp.where(iota < 2, jnp.int32(2 * PC + 1), iota))
    stage.append(pltpu.async_copy(small_hbm.at[idx_s], gbufA.at[0].at[pl.ds(0, 16)], gsemA))
    for cp in stage:
        cp.wait()
    sm = [[gbufA[0, ch, pl.ds(fg * 16, 16)] for fg in range(4)] for ch in range(2)]

    def fold_body(j, carry):
        for ch in range(2):
            for fg in range(4):
                fs = pl.ds(fg * 16, 16)
                mpan[ch, j, fs] = mpan[ch, j, fs] + sm[ch][fg]
        return carry
    lax.fori_loop(0, CELLS, fold_body, 0)

    def build_idx_A(slot):
        for g in range(8):
            s0 = pre2[slot, pl.ds(g * 16, 16)]
            s1 = pre2[slot, pl.ds(CPAD + g * 16, 16)] + jnp.int32(PC + 1)
            base = (iota + g * 16) * ED
            idxA[0, pl.ds(g * 16, 16)] = s0 + base
            idxA[1, pl.ds(g * 16, 16)] = s1 + base
            idxA[2, pl.ds(g * 16, 16)] = s0
            idxA[3, pl.ds(g * 16, 16)] = s1

    def build_idx_B(slot):
        for g in range(7):
            gg = g + 8
            c = iota + gg * 16
            valid = c < CELLS
            s0 = jnp.where(valid, pre2[slot, pl.ds(gg * 16, 16)], wid)
            s1 = jnp.where(valid, pre2[slot, pl.ds(CPAD + gg * 16, 16)], wid) + jnp.int32(PC + 1)
            base = jnp.where(valid, c, c - 128) * ED
            idxB[0, pl.ds(g * 16, 16)] = s0 + base
            idxB[1, pl.ds(g * 16, 16)] = s1 + base
            idxB[2, pl.ds(g * 16, 16)] = s0
            idxB[3, pl.ds(g * 16, 16)] = s1

    def fire(idx, gbuf, sem):
        pltpu.async_copy(big_hbm.at[idx.at[0]], gbuf.at[0], sem)
        pltpu.async_copy(big_hbm.at[idx.at[1]], gbuf.at[1], sem)
        pltpu.async_copy(small_hbm.at[idx.at[2]], gbuf.at[2], sem)
        pltpu.async_copy(small_hbm.at[idx.at[3]], gbuf.at[3], sem)

    def drain(idx, gbuf, sem):
        pltpu.make_async_copy(big_hbm.at[idx.at[0]], gbuf.at[0], sem).wait()
        pltpu.make_async_copy(big_hbm.at[idx.at[1]], gbuf.at[1], sem).wait()
        pltpu.make_async_copy(small_hbm.at[idx.at[2]], gbuf.at[2], sem).wait()
        pltpu.make_async_copy(small_hbm.at[idx.at[3]], gbuf.at[3], sem).wait()

    def build_mask(slot, groups, local_off):
        for g in groups:
            bd0 = pre2[slot, pl.ds(2 * CPAD + g * 16, 16)]
            bd1 = pre2[slot, pl.ds(3 * CPAD + g * 16, 16)]
            mv0 = jnp.where(bd0 > 0, jnp.float32(1.0), jnp.float32(0.0))
            mv1 = jnp.where(bd1 > 0, jnp.float32(1.0), jnp.float32(0.0))
            cbase = (iota + (g * 16 - local_off)) * 16
            for l in range(16):
                plsc.store_scatter(mskb.at[0], [cbase + l], mv0)
                plsc.store_scatter(mskb.at[1], [cbase + l], mv1)

    def transpose_A(sel):
        obase = sel * SLAB
        def body(r, cr):
            m0 = mskb[0, pl.ds(r * 16, 16)] > jnp.float32(0.5)
            m1 = mskb[1, pl.ds(r * 16, 16)] > jnp.float32(0.5)
            for fg in range(4):
                fs = pl.ds(fg * 16, 16)
                v0 = jnp.where(m0, mpan[0, r, fs], gbufA[0, r, fs] + gbufA[2, r, fs])
                v1 = jnp.where(m1, mpan[1, r, fs], gbufA[1, r, fs] + gbufA[3, r, fs])
                plsc.store_scatter(out2, [fidx[fg] + (r + obase)], v0 + v1)
            return cr
        lax.fori_loop(0, 128, body, 0)

    def transpose_B(sel):
        obase = sel * SLAB + 128
        def body(r, cr):
            c = 128 + r
            m0 = mskb[0, pl.ds(r * 16, 16)] > jnp.float32(0.5)
            m1 = mskb[1, pl.ds(r * 16, 16)] > jnp.float32(0.5)
            for fg in range(4):
                fs = pl.ds(fg * 16, 16)
                v0 = jnp.where(m0, mpan[0, c, fs], gbufB[0, r, fs] + gbufB[2, r, fs])
                v1 = jnp.where(m1, mpan[1, c, fs], gbufB[1, r, fs] + gbufB[3, r, fs])
                plsc.store_scatter(out2, [fidx[fg] + (r + obase)], v0 + v1)
            return cr
        lax.fori_loop(0, CELLS - 128, body, 0)

    pltpu.sync_copy(pre_hbm.at[b_base], pre2.at[0])
    pltpu.async_copy(pre_hbm.at[b_base + 1], pre2.at[1], psem)
    build_idx_A(0)
    fire(idxA, gbufA, gsemA)
    build_mask(0, range(8), 0)

    def per_batch(i, carry):
        b = b_base + i
        slot = lax.rem(i, 2)
        nslot = lax.rem(i + 1, 2)
        sel = slot

        build_idx_B(slot)
        fire(idxB, gbufB, gsemB)

        drain(idxA, gbufA, gsemA)
        transpose_A(sel)
        build_mask(slot, range(8, 15), 128)

        pltpu.make_async_copy(pre_hbm.at[b], pre2.at[nslot], psem).wait()
        nxt = b_base + jnp.minimum(i + 2, bpw - 1)
        pltpu.async_copy(pre_hbm.at[nxt], pre2.at[slot], psem)

        @pl.when(i < bpw - 1)
        def _fire_next_a():
            build_idx_A(nslot)
            fire(idxA, gbufA, gsemA)

        drain(idxB, gbufB, gsemB)
        transpose_B(sel)

        @pl.when(i < bpw - 1)
        def _mask_next_a():
            build_mask(nslot, range(8), 0)

        @pl.when(i >= 1)
        def _drain_out():
            pltpu.make_async_copy(out2.at[pl.ds(sel * SLAB, SLAB)], out_hbm.at[b], osem).wait()
        pltpu.async_copy(out2.at[pl.ds(sel * SLAB, SLAB)], out_hbm.at[b], osem)
        return carry

    lax.fori_loop(0, bpw, per_batch, 0)

    pltpu.make_async_copy(pre_hbm.at[b_base], pre2.at[0], psem).wait()
    pltpu.make_async_copy(out2.at[pl.ds(0, SLAB)], out_hbm.at[b_base + bpw - 1], osem).wait()


def kernel(sparse_feature_dim, sparse_feature_input, board_input, pcode_table, pcode_board_table):
    del sparse_feature_dim
    s = sparse_feature_input[:, 10:12].reshape(B, 2, CELLS)
    bd = board_input.reshape(B, 2, CELLS)
    packed = jnp.pad(jnp.concatenate([s, bd], axis=1),
                     ((0, 0), (0, 0), (0, CPAD - CELLS))).reshape(B, 4 * CPAD)

    mesh = plsc.VectorSubcoreMesh(core_axis_name="c", subcore_axis_name="s")
    out = pl.kernel(
        _sc_kernel,
        out_type=jax.ShapeDtypeStruct((B, SLAB), jnp.float32),
        mesh=mesh,
        compiler_params=pltpu.CompilerParams(
            needs_layout_passes=False, use_tc_tiling_on_sc=False),
        scratch_types=[
            pltpu.VMEM((2, 4 * CPAD), jnp.int32),
            pltpu.VMEM((4, 128), jnp.int32),
            pltpu.VMEM((4, NB), jnp.int32),
            pltpu.VMEM((16,), jnp.int32),
            pltpu.VMEM((4, 128, FD), jnp.float32),
            pltpu.VMEM((4, NB, FD), jnp.float32),
            pltpu.VMEM((2, 240, FD), jnp.float32),
            pltpu.VMEM((2, 2048), jnp.float32),
            pltpu.VMEM((2 * SLAB,), jnp.float32),
            pltpu.SemaphoreType.DMA,
            pltpu.SemaphoreType.DMA,
            pltpu.SemaphoreType.DMA,
            pltpu.SemaphoreType.DMA,
        ],
    )(packed, pcode_board_table, pcode_table)
    return out.reshape(B, FD, BS, BS)


# speedup vs baseline: 3.3963x; 1.0233x over previous
import jax
import jax.numpy as jnp
from jax import lax
from jax.experimental import pallas as pl
from jax.experimental.pallas import tpu as pltpu
from jax.experimental.pallas import tpu_sc as plsc

B = 1024
BS = 15
FD = 64
PC = 2380
ED = 2 * (PC + 1)
CELLS = BS * BS
CPAD = 240
NB = 112
SLAB = FD * CELLS


def _sc_kernel(pre_hbm, big_hbm, small_hbm, out_hbm,
               pre2, idxA, idxB, idx_s, gbufA, gbufB, mpan, mskb, out2,
               gsemA, gsemB, psem, osem):
    info = plsc.get_sparse_core_info()
    nc = info.num_cores
    wid = lax.axis_index("s") * nc + lax.axis_index("c")
    bpw = B // (nc * info.num_subcores)
    b_base = wid * bpw

    iota = lax.iota(jnp.int32, 16)
    fidx = [(iota + fg * 16) * CELLS for fg in range(4)]

    for g in range(8):
        c = iota + g * 16
        idxA[0, pl.ds(g * 16, 16)] = c * ED + jnp.int32(PC)
        idxA[1, pl.ds(g * 16, 16)] = c * ED + jnp.int32(2 * PC + 1)
    for g in range(7):
        c = iota + (g + 8) * 16
        c_eff = jnp.where(c < CELLS, c, c - 128)
        idxB[0, pl.ds(g * 16, 16)] = c_eff * ED + jnp.int32(PC)
        idxB[1, pl.ds(g * 16, 16)] = c_eff * ED + jnp.int32(2 * PC + 1)
    stage = [pltpu.async_copy(big_hbm.at[idxA.at[0]], mpan.at[0].at[pl.ds(0, 128)], gsemA),
             pltpu.async_copy(big_hbm.at[idxA.at[1]], mpan.at[1].at[pl.ds(0, 128)], gsemA),
             pltpu.async_copy(big_hbm.at[idxB.at[0]], gbufB.at[0], gsemB),
             pltpu.async_copy(big_hbm.at[idxB.at[1]], gbufB.at[1], gsemB)]
    idx_s[pl.ds(0, 16)] = jnp.where(iota < 1, jnp.int32(PC),
                                    jnp.where(iota < 2, jnp.int32(2 * PC + 1), iota))
    stage.append(pltpu.async_copy(small_hbm.at[idx_s], gbufA.at[0].at[pl.ds(0, 16)], gsemA))
    for cp in stage:
        cp.wait()
    sm = [[gbufA[0, ch, pl.ds(fg * 16, 16)] for fg in range(4)] for ch in range(2)]

    def fold_a(j, carry):
        for ch in range(2):
            for fg in range(4):
                fs = pl.ds(fg * 16, 16)
                mpan[ch, j, fs] = mpan[ch, j, fs] + sm[ch][fg]
        return carry
    lax.fori_loop(0, 128, fold_a, 0)

    def fold_b(r, carry):
        for ch in range(2):
            for fg in range(4):
                fs = pl.ds(fg * 16, 16)
                mpan[ch, 128 + r, fs] = gbufB[ch, r, fs] + sm[ch][fg]
        return carry
    lax.fori_loop(0, CELLS - 128, fold_b, 0)

    def build_idx_A(slot):
        for g in range(8):
            s0 = pre2[slot, pl.ds(g * 16, 16)]
            s1 = pre2[slot, pl.ds(CPAD + g * 16, 16)] + jnp.int32(PC + 1)
            base = (iota + g * 16) * ED
            idxA[0, pl.ds(g * 16, 16)] = s0 + base
            idxA[1, pl.ds(g * 16, 16)] = s1 + base
            idxA[2, pl.ds(g * 16, 16)] = s0
            idxA[3, pl.ds(g * 16, 16)] = s1

    def build_idx_B(slot):
        for g in range(7):
            gg = g + 8
            c = iota + gg * 16
            valid = c < CELLS
            s0 = jnp.where(valid, pre2[slot, pl.ds(gg * 16, 16)], wid)
            s1 = jnp.where(valid, pre2[slot, pl.ds(CPAD + gg * 16, 16)], wid) + jnp.int32(PC + 1)
            base = jnp.where(valid, c, c - 128) * ED
            idxB[0, pl.ds(g * 16, 16)] = s0 + base
            idxB[1, pl.ds(g * 16, 16)] = s1 + base
            idxB[2, pl.ds(g * 16, 16)] = s0
            idxB[3, pl.ds(g * 16, 16)] = s1

    def fire(idx, gbuf, sem):
        pltpu.async_copy(big_hbm.at[idx.at[0]], gbuf.at[0], sem)
        pltpu.async_copy(big_hbm.at[idx.at[1]], gbuf.at[1], sem)
        pltpu.async_copy(small_hbm.at[idx.at[2]], gbuf.at[2], sem)
        pltpu.async_copy(small_hbm.at[idx.at[3]], gbuf.at[3], sem)

    def drain(idx, gbuf, sem):
        pltpu.make_async_copy(big_hbm.at[idx.at[0]], gbuf.at[0], sem).wait()
        pltpu.make_async_copy(big_hbm.at[idx.at[1]], gbuf.at[1], sem).wait()
        pltpu.make_async_copy(small_hbm.at[idx.at[2]], gbuf.at[2], sem).wait()
        pltpu.make_async_copy(small_hbm.at[idx.at[3]], gbuf.at[3], sem).wait()

    def build_mask(slot, groups, local_off):
        for g in groups:
            bd0 = pre2[slot, pl.ds(2 * CPAD + g * 16, 16)]
            bd1 = pre2[slot, pl.ds(3 * CPAD + g * 16, 16)]
            mv = (jnp.where(bd0 > 0, jnp.int32(1), jnp.int32(0))
                  + jnp.where(bd1 > 0, jnp.int32(2), jnp.int32(0)))
            cbase = (iota + (g * 16 - local_off)) * 16
            for l in range(16):
                plsc.store_scatter(mskb, [cbase + l], mv)

    def transpose_A():
        def body(r, cr):
            mr = mskb[pl.ds(r * 16, 16)]
            m0 = jnp.bitwise_and(mr, 1) > 0
            m1 = jnp.bitwise_and(mr, 2) > 0
            for fg in range(4):
                fs = pl.ds(fg * 16, 16)
                v0 = jnp.where(m0, mpan[0, r, fs], gbufA[0, r, fs] + gbufA[2, r, fs])
                v1 = jnp.where(m1, mpan[1, r, fs], gbufA[1, r, fs] + gbufA[3, r, fs])
                plsc.store_scatter(out2, [fidx[fg] + r], v0 + v1)
            return cr
        lax.fori_loop(0, 128, body, 0)

    def transpose_B():
        def body(r, cr):
            c = 128 + r
            mr = mskb[pl.ds(r * 16, 16)]
            m0 = jnp.bitwise_and(mr, 1) > 0
            m1 = jnp.bitwise_and(mr, 2) > 0
            for fg in range(4):
                fs = pl.ds(fg * 16, 16)
                v0 = jnp.where(m0, mpan[0, c, fs], gbufB[0, r, fs] + gbufB[2, r, fs])
                v1 = jnp.where(m1, mpan[1, c, fs], gbufB[1, r, fs] + gbufB[3, r, fs])
                plsc.store_scatter(out2, [fidx[fg] + c], v0 + v1)
            return cr
        lax.fori_loop(0, CELLS - 128, body, 0)

    pltpu.sync_copy(pre_hbm.at[b_base], pre2.at[0])
    pltpu.async_copy(pre_hbm.at[b_base + 1], pre2.at[1], psem)
    build_idx_A(0)
    fire(idxA, gbufA, gsemA)
    build_mask(0, range(8), 0)

    def per_batch(i, carry):
        b = b_base + i
        slot = lax.rem(i, 2)
        nslot = lax.rem(i + 1, 2)

        build_idx_B(slot)
        fire(idxB, gbufB, gsemB)

        @pl.when(i >= 1)
        def _drain_out():
            pltpu.make_async_copy(out2, out_hbm.at[b], osem).wait()

        drain(idxA, gbufA, gsemA)
        transpose_A()
        build_mask(slot, range(8, 15), 128)

        pltpu.make_async_copy(pre_hbm.at[b], pre2.at[nslot], psem).wait()
        nxt = b_base + jnp.minimum(i + 2, bpw - 1)
        pltpu.async_copy(pre_hbm.at[nxt], pre2.at[slot], psem)

        @pl.when(i < bpw - 1)
        def _fire_next_a():
            build_idx_A(nslot)
            fire(idxA, gbufA, gsemA)

        drain(idxB, gbufB, gsemB)
        transpose_B()

        @pl.when(i < bpw - 1)
        def _mask_next_a():
            build_mask(nslot, range(8), 0)

        pltpu.async_copy(out2, out_hbm.at[b], osem)
        return carry

    lax.fori_loop(0, bpw, per_batch, 0)

    pltpu.make_async_copy(pre_hbm.at[b_base], pre2.at[0], psem).wait()
    pltpu.make_async_copy(out2, out_hbm.at[b_base + bpw - 1], osem).wait()


def kernel(sparse_feature_dim, sparse_feature_input, board_input, pcode_table, pcode_board_table):
    del sparse_feature_dim
    s = sparse_feature_input[:, 10:12].reshape(B, 2, CELLS)
    bd = board_input.reshape(B, 2, CELLS)
    packed = jnp.pad(jnp.concatenate([s, bd], axis=1),
                     ((0, 0), (0, 0), (0, CPAD - CELLS))).reshape(B, 4 * CPAD)

    mesh = plsc.VectorSubcoreMesh(core_axis_name="c", subcore_axis_name="s")
    out = pl.kernel(
        _sc_kernel,
        out_type=jax.ShapeDtypeStruct((B, SLAB), jnp.float32),
        mesh=mesh,
        compiler_params=pltpu.CompilerParams(
            needs_layout_passes=False, use_tc_tiling_on_sc=False),
        scratch_types=[
            pltpu.VMEM((2, 4 * CPAD), jnp.int32),
            pltpu.VMEM((4, 128), jnp.int32),
            pltpu.VMEM((4, NB), jnp.int32),
            pltpu.VMEM((16,), jnp.int32),
            pltpu.VMEM((4, 128, FD), jnp.float32),
            pltpu.VMEM((4, NB, FD), jnp.float32),
            pltpu.VMEM((2, CELLS, FD), jnp.float32),
            pltpu.VMEM((2048,), jnp.int32),
            pltpu.VMEM((SLAB,), jnp.float32),
            pltpu.SemaphoreType.DMA,
            pltpu.SemaphoreType.DMA,
            pltpu.SemaphoreType.DMA,
            pltpu.SemaphoreType.DMA,
        ],
    )(packed, pcode_board_table, pcode_table)
    return out.reshape(B, FD, BS, BS)


# speedup vs baseline: 3.3978x; 1.0005x over previous
import jax
import jax.numpy as jnp
from jax import lax
from jax.experimental import pallas as pl
from jax.experimental.pallas import tpu as pltpu
from jax.experimental.pallas import tpu_sc as plsc

B = 1024
BS = 15
FD = 64
PC = 2380
ED = 2 * (PC + 1)
CELLS = BS * BS
CPAD = 240
NB = 112
NTAIL = 97
SLAB = FD * CELLS


def _sc_kernel(pre_hbm, big_hbm, small_hbm, out_hbm,
               pre2, idxA, idxB, idx_s, gbufA, gbufB, mpan, mskb, out2,
               gsemA, gsemB, psem, osem):
    info = plsc.get_sparse_core_info()
    nc = info.num_cores
    wid = lax.axis_index("s") * nc + lax.axis_index("c")
    bpw = B // (nc * info.num_subcores)
    b_base = wid * bpw

    iota = lax.iota(jnp.int32, 16)
    fidx = [(iota + fg * 16) * CELLS for fg in range(4)]

    for g in range(8):
        c = iota + g * 16
        idxA[0, pl.ds(g * 16, 16)] = c * ED + jnp.int32(PC)
        idxA[1, pl.ds(g * 16, 16)] = c * ED + jnp.int32(2 * PC + 1)
    for g in range(7):
        c = iota + (g + 8) * 16
        c_eff = jnp.where(c < CELLS, c, c - 128)
        idxB[0, pl.ds(g * 16, 16)] = c_eff * ED + jnp.int32(PC)
        idxB[1, pl.ds(g * 16, 16)] = c_eff * ED + jnp.int32(2 * PC + 1)
    stage = [pltpu.async_copy(big_hbm.at[idxA.at[0]], mpan.at[0].at[pl.ds(0, 128)], gsemA),
             pltpu.async_copy(big_hbm.at[idxA.at[1]], mpan.at[1].at[pl.ds(0, 128)], gsemA),
             pltpu.async_copy(big_hbm.at[idxB.at[0].at[pl.ds(0, NTAIL)]], gbufB.at[0], gsemB),
             pltpu.async_copy(big_hbm.at[idxB.at[1].at[pl.ds(0, NTAIL)]], gbufB.at[1], gsemB)]
    idx_s[pl.ds(0, 16)] = jnp.where(iota < 1, jnp.int32(PC),
                                    jnp.where(iota < 2, jnp.int32(2 * PC + 1), iota))
    stage.append(pltpu.async_copy(small_hbm.at[idx_s], gbufA.at[0].at[pl.ds(0, 16)], gsemA))
    for cp in stage:
        cp.wait()
    sm = [[gbufA[0, ch, pl.ds(fg * 16, 16)] for fg in range(4)] for ch in range(2)]

    def fold_a(j, carry):
        for ch in range(2):
            for fg in range(4):
                fs = pl.ds(fg * 16, 16)
                mpan[ch, j, fs] = mpan[ch, j, fs] + sm[ch][fg]
        return carry
    lax.fori_loop(0, 128, fold_a, 0)

    def fold_b(r, carry):
        for ch in range(2):
            for fg in range(4):
                fs = pl.ds(fg * 16, 16)
                mpan[ch, 128 + r, fs] = gbufB[ch, r, fs] + sm[ch][fg]
        return carry
    lax.fori_loop(0, CELLS - 128, fold_b, 0)

    def build_idx_A(slot):
        for g in range(8):
            s0 = pre2[slot, pl.ds(g * 16, 16)]
            s1 = pre2[slot, pl.ds(CPAD + g * 16, 16)] + jnp.int32(PC + 1)
            base = (iota + g * 16) * ED
            idxA[0, pl.ds(g * 16, 16)] = s0 + base
            idxA[1, pl.ds(g * 16, 16)] = s1 + base
            idxA[2, pl.ds(g * 16, 16)] = s0
            idxA[3, pl.ds(g * 16, 16)] = s1

    def build_idx_B(slot):
        for g in range(7):
            gg = g + 8
            c = iota + gg * 16
            valid = c < CELLS
            s0 = jnp.where(valid, pre2[slot, pl.ds(gg * 16, 16)], wid)
            s1 = jnp.where(valid, pre2[slot, pl.ds(CPAD + gg * 16, 16)], wid) + jnp.int32(PC + 1)
            base = jnp.where(valid, c, c - 128) * ED
            idxB[0, pl.ds(g * 16, 16)] = s0 + base
            idxB[1, pl.ds(g * 16, 16)] = s1 + base
            idxB[2, pl.ds(g * 16, 16)] = s0
            idxB[3, pl.ds(g * 16, 16)] = s1

    def _idx4(idx, n):
        if n == 128:
            return [idx.at[k] for k in range(4)]
        return [idx.at[k].at[pl.ds(0, n)] for k in range(4)]

    def fire(idx, gbuf, sem, n):
        i4 = _idx4(idx, n)
        pltpu.async_copy(big_hbm.at[i4[0]], gbuf.at[0], sem)
        pltpu.async_copy(big_hbm.at[i4[1]], gbuf.at[1], sem)
        pltpu.async_copy(small_hbm.at[i4[2]], gbuf.at[2], sem)
        pltpu.async_copy(small_hbm.at[i4[3]], gbuf.at[3], sem)

    def drain(idx, gbuf, sem, n):
        i4 = _idx4(idx, n)
        pltpu.make_async_copy(big_hbm.at[i4[0]], gbuf.at[0], sem).wait()
        pltpu.make_async_copy(big_hbm.at[i4[1]], gbuf.at[1], sem).wait()
        pltpu.make_async_copy(small_hbm.at[i4[2]], gbuf.at[2], sem).wait()
        pltpu.make_async_copy(small_hbm.at[i4[3]], gbuf.at[3], sem).wait()

    def build_mask(slot, groups, local_off):
        for g in groups:
            bd0 = pre2[slot, pl.ds(2 * CPAD + g * 16, 16)]
            bd1 = pre2[slot, pl.ds(3 * CPAD + g * 16, 16)]
            mv = (jnp.where(bd0 > 0, jnp.int32(1), jnp.int32(0))
                  + jnp.where(bd1 > 0, jnp.int32(2), jnp.int32(0)))
            cbase = (iota + (g * 16 - local_off)) * 16
            for l in range(16):
                plsc.store_scatter(mskb, [cbase + l], mv)

    def transpose_A():
        def body(r, cr):
            mr = mskb[pl.ds(r * 16, 16)]
            m0 = jnp.bitwise_and(mr, 1) > 0
            m1 = jnp.bitwise_and(mr, 2) > 0
            for fg in range(4):
                fs = pl.ds(fg * 16, 16)
                v0 = jnp.where(m0, mpan[0, r, fs], gbufA[0, r, fs] + gbufA[2, r, fs])
                v1 = jnp.where(m1, mpan[1, r, fs], gbufA[1, r, fs] + gbufA[3, r, fs])
                plsc.store_scatter(out2, [fidx[fg] + r], v0 + v1)
            return cr
        lax.fori_loop(0, 128, body, 0)

    def transpose_B():
        def body(r, cr):
            c = 128 + r
            mr = mskb[pl.ds(r * 16, 16)]
            m0 = jnp.bitwise_and(mr, 1) > 0
            m1 = jnp.bitwise_and(mr, 2) > 0
            for fg in range(4):
                fs = pl.ds(fg * 16, 16)
                v0 = jnp.where(m0, mpan[0, c, fs], gbufB[0, r, fs] + gbufB[2, r, fs])
                v1 = jnp.where(m1, mpan[1, c, fs], gbufB[1, r, fs] + gbufB[3, r, fs])
                plsc.store_scatter(out2, [fidx[fg] + c], v0 + v1)
            return cr
        lax.fori_loop(0, CELLS - 128, body, 0)

    pltpu.sync_copy(pre_hbm.at[b_base], pre2.at[0])
    pltpu.async_copy(pre_hbm.at[b_base + 1], pre2.at[1], psem)
    build_idx_A(0)
    fire(idxA, gbufA, gsemA, 128)
    build_mask(0, range(8), 0)

    def per_batch(i, carry):
        b = b_base + i
        slot = lax.rem(i, 2)
        nslot = lax.rem(i + 1, 2)

        build_idx_B(slot)
        fire(idxB, gbufB, gsemB, NTAIL)

        @pl.when(i >= 1)
        def _drain_out():
            pltpu.make_async_copy(out2, out_hbm.at[b], osem).wait()

        drain(idxA, gbufA, gsemA, 128)
        transpose_A()
        build_mask(slot, range(8, 15), 128)

        pltpu.make_async_copy(pre_hbm.at[b], pre2.at[nslot], psem).wait()
        nxt = b_base + jnp.minimum(i + 2, bpw - 1)
        pltpu.async_copy(pre_hbm.at[nxt], pre2.at[slot], psem)

        @pl.when(i < bpw - 1)
        def _fire_next_a():
            build_idx_A(nslot)
            fire(idxA, gbufA, gsemA, 128)

        drain(idxB, gbufB, gsemB, NTAIL)
        transpose_B()

        @pl.when(i < bpw - 1)
        def _mask_next_a():
            build_mask(nslot, range(8), 0)

        pltpu.async_copy(out2, out_hbm.at[b], osem)
        return carry

    lax.fori_loop(0, bpw, per_batch, 0)

    pltpu.make_async_copy(pre_hbm.at[b_base], pre2.at[0], psem).wait()
    pltpu.make_async_copy(out2, out_hbm.at[b_base + bpw - 1], osem).wait()


def kernel(sparse_feature_dim, sparse_feature_input, board_input, pcode_table, pcode_board_table):
    del sparse_feature_dim
    s = sparse_feature_input[:, 10:12].reshape(B, 2, CELLS)
    bd = board_input.reshape(B, 2, CELLS)
    packed = jnp.pad(jnp.concatenate([s, bd], axis=1),
                     ((0, 0), (0, 0), (0, CPAD - CELLS))).reshape(B, 4 * CPAD)

    mesh = plsc.VectorSubcoreMesh(core_axis_name="c", subcore_axis_name="s")
    out = pl.kernel(
        _sc_kernel,
        out_type=jax.ShapeDtypeStruct((B, SLAB), jnp.float32),
        mesh=mesh,
        compiler_params=pltpu.CompilerParams(
            needs_layout_passes=False, use_tc_tiling_on_sc=False),
        scratch_types=[
            pltpu.VMEM((2, 4 * CPAD), jnp.int32),
            pltpu.VMEM((4, 128), jnp.int32),
            pltpu.VMEM((4, NB), jnp.int32),
            pltpu.VMEM((16,), jnp.int32),
            pltpu.VMEM((4, 128, FD), jnp.float32),
            pltpu.VMEM((4, NTAIL, FD), jnp.float32),
            pltpu.VMEM((2, CELLS, FD), jnp.float32),
            pltpu.VMEM((2048,), jnp.int32),
            pltpu.VMEM((SLAB,), jnp.float32),
            pltpu.SemaphoreType.DMA,
            pltpu.SemaphoreType.DMA,
            pltpu.SemaphoreType.DMA,
            pltpu.SemaphoreType.DMA,
        ],
    )(packed, pcode_board_table, pcode_table)
    return out.reshape(B, FD, BS, BS)
